# baseline (speedup 1.0000x reference)
import functools

import jax
import jax.numpy as jnp
from jax import lax
from jax.experimental import pallas as pl
from jax.experimental.pallas import tpu as pltpu
from jax.experimental.pallas import tpu_sc as plsc

NU = 10000
NI = 10000
NT = 2000
NG = NU + NI + NT
D = 768

CW = 64
NCHUNK = D // CW
NC = 2
NS = 16
EB = 128
_GDN = lax.GatherDimensionNumbers(offset_dims=(), collapsed_slice_dims=(0,),
                                  start_index_map=(0,))
ZROWS = 125


def _chunked(x):
    n = x.shape[0]
    return jnp.transpose(x.reshape(n, NCHUNK, CW), (1, 0, 2))


def _pad_edges(idx, val, n_rows, n_cols):
    nnz = val.shape[0]
    e_pad = -(-nnz // (NS * EB * 2)) * (NS * EB * 2)
    pad = e_pad - nnz
    ar = jnp.arange(pad, dtype=jnp.int32)
    rows = jnp.concatenate([idx[0], ar % n_rows])
    cols = jnp.concatenate([idx[1], ar % n_cols])
    vals = jnp.concatenate([val, jnp.zeros((pad,), jnp.float32)])
    return rows, cols, vals


@functools.lru_cache(maxsize=None)
def _make_spmm(n_rows, n_cols, e_pad):
    per_tile = e_pad // NS
    n_batches = per_tile // EB
    chunks_per_sc = NCHUNK // NC
    stripe = -(-n_rows // (8 * NS)) * 8
    stripe_last = n_rows - (NS - 1) * stripe
    assert stripe_last > 0 and stripe_last % 8 == 0
    mesh = plsc.VectorSubcoreMesh(core_axis_name="c", subcore_axis_name="s",
                                  num_cores=NC, num_subcores=NS)

    @functools.partial(
        pl.kernel,
        out_type=jax.ShapeDtypeStruct((NCHUNK, n_rows, CW), jnp.float32),
        mesh=mesh,
        scratch_types=[
            pltpu.VMEM_SHARED((n_rows, CW), jnp.float32),
            pltpu.VMEM((n_batches, EB), jnp.int32),
            pltpu.VMEM((n_batches, EB), jnp.int32),
            pltpu.VMEM((per_tile + 16,), jnp.float32),
            pltpu.VMEM((EB, CW), jnp.float32),
            pltpu.VMEM((EB, CW), jnp.float32),
            pltpu.SemaphoreType.DMA,
            pltpu.SemaphoreType.DMA,
            pltpu.SemaphoreType.DMA,
            pltpu.SemaphoreType.DMA,
        ],
        compiler_params=pltpu.CompilerParams(use_tc_tiling_on_sc=False),
    )
    def spmm(x_hbm, rows_hbm, cols_hbm, vals_hbm, zeros_hbm, out_hbm,
             acc, colsl, rowsl, valsl, gbuf0, gbuf1,
             semg0, semg1, sems0, sems1):
        cid = lax.axis_index("c")
        sid = lax.axis_index("s")
        rbase = sid * stripe

        tb0 = sid * n_batches
        pltpu.sync_copy(cols_hbm.at[pl.ds(tb0, n_batches), :], colsl)
        pltpu.sync_copy(rows_hbm.at[pl.ds(tb0, n_batches), :], rowsl)
        pltpu.sync_copy(vals_hbm.at[pl.ds(sid * per_tile, per_tile)],
                        valsl.at[pl.ds(0, per_tile)])

        gbuf = [gbuf0, gbuf1]
        semg = [semg0, semg1]
        sems = [sems0, sems1]

        @pl.loop(0, chunks_per_sc)
        def _(ci):
            c = cid * chunks_per_sc + ci

            @pl.when(sid < NS - 1)
            def _():
                pltpu.sync_copy(zeros_hbm.at[pl.ds(0, stripe), :],
                                acc.at[pl.ds(rbase, stripe), :])

            @pl.when(sid == NS - 1)
            def _():
                pltpu.sync_copy(zeros_hbm.at[pl.ds(0, stripe_last), :],
                                acc.at[pl.ds(rbase, stripe_last), :])

            plsc.subcore_barrier()

            def scale(buf, b):
                @pl.loop(0, EB // 16)
                def _(g):
                    gv = valsl[pl.ds(b * EB + g * 16, 16)]
                    for lane in range(16):
                        bv = lax.gather(
                            gv, jnp.full((16, 1), lane, jnp.int32),
                            _GDN, slice_sizes=(1,),
                            mode=lax.GatherScatterMode.PROMISE_IN_BOUNDS)
                        e = g * 16 + lane
                        for j in range(CW // 16):
                            buf[e, pl.ds(16 * j, 16)] = (
                                buf[e, pl.ds(16 * j, 16)] * bv)

            def gather(b, k):
                return pltpu.async_copy(x_hbm.at[c].at[colsl.at[b]],
                                        gbuf[k], semg[k])

            def wait_gather(b, k):
                pltpu.make_async_copy(x_hbm.at[c].at[colsl.at[b]],
                                      gbuf[k], semg[k]).wait()

            def scatter(b, k):
                return pltpu.async_copy(gbuf[k], acc.at[rowsl.at[b]],
                                        sems[k], add=True)

            def wait_scatter(b, k):
                pltpu.make_async_copy(gbuf[k], acc.at[rowsl.at[b]],
                                      sems[k]).wait()

            gather(0, 0)

            @pl.loop(0, n_batches // 2)
            def _(p):
                b0 = 2 * p

                @pl.when(p > 0)
                def _():
                    wait_scatter(b0 - 1, 1)

                gather(b0 + 1, 1)
                wait_gather(b0, 0)
                scale(gbuf[0], b0)
                scatter(b0, 0)
                wait_gather(b0 + 1, 1)
                scale(gbuf[1], b0 + 1)

                @pl.when(p < n_batches // 2 - 1)
                def _():
                    wait_scatter(b0, 0)
                    gather(b0 + 2, 0)

                scatter(b0 + 1, 1)

            wait_scatter(n_batches - 2, 0)
            wait_scatter(n_batches - 1, 1)

            plsc.subcore_barrier()

            @pl.when(sid < NS - 1)
            def _():
                pltpu.sync_copy(acc.at[pl.ds(rbase, stripe), :],
                                out_hbm.at[c, pl.ds(rbase, stripe), :])

            @pl.when(sid == NS - 1)
            def _():
                pltpu.sync_copy(acc.at[pl.ds(rbase, stripe_last), :],
                                out_hbm.at[c, pl.ds(rbase, stripe_last), :])

            plsc.subcore_barrier()

    return spmm


def _spmm(idx, val, x_chunks, n_rows, n_cols):
    rows, cols, vals = _pad_edges(idx, val, n_rows, n_cols)
    e_pad = int(vals.shape[0])
    stripe = -(-n_rows // (8 * NS)) * 8
    zeros = jnp.zeros((stripe, CW), jnp.float32)
    fn = _make_spmm(n_rows, n_cols, e_pad)
    return fn(x_chunks, rows.reshape(e_pad // EB, EB),
              cols.reshape(e_pad // EB, EB), vals, zeros)


def _gelu_exact(x):
    return 0.5 * x * (1.0 + lax.erf(x * (2.0 ** -0.5)))


def _mlp_block(x, w1, b1, g, be, w2, b2):
    h = jnp.dot(x, w1, preferred_element_type=jnp.float32) + b1
    m = jnp.mean(h, axis=1, keepdims=True)
    v = jnp.mean((h - m) * (h - m), axis=1, keepdims=True)
    h = (h - m) * lax.rsqrt(v + 1e-5) * g + be
    h = _gelu_exact(h)
    return jnp.dot(h, w2, preferred_element_type=jnp.float32) + b2


BLK = 200
NBLK = NU // BLK


@functools.lru_cache(maxsize=None)
def _make_dense_side(g_row_off):
    grid = (NBLK,)

    def body(uw_s, tw_s, vw_s, lgw_s, gw_s,
             ci_ref, cp_ref, ti_ref, tp_ref, gi_ref, gp_ref,
             w1l, b1l, gl, bel, w2l, b2l,
             w1g, b1g, gg, beg, w2g, b2g,
             pw, pb,
             out_final, out_cp, out_tp, out_lp, out_gp, out_logit):
        call_emb = uw_s[0] * ci_ref[...] + uw_s[1] * cp_ref[...]
        tag_emb = tw_s[0] * ti_ref[...] + tw_s[1] * tp_ref[...]
        g_emb = gw_s[0] * gi_ref[...] + gw_s[1] * gp_ref[...]
        mixed = vw_s[0] * call_emb + vw_s[1] * tag_emb
        final = lgw_s[0] * mixed + lgw_s[1] * g_emb
        out_final[...] = final
        xs = jnp.concatenate([call_emb, tag_emb, mixed], axis=0)
        ys = _mlp_block(xs, w1l[...], b1l[...], gl[...], bel[...],
                        w2l[...], b2l[...])
        out_cp[...] = ys[:BLK]
        out_tp[...] = ys[BLK:2 * BLK]
        out_lp[...] = ys[2 * BLK:]
        out_gp[...] = _mlp_block(g_emb, w1g[...], b1g[...], gg[...], beg[...],
                                 w2g[...], b2g[...])
        out_logit[...] = (jnp.dot(final, pw[...], preferred_element_type=jnp.float32)
                          + pb[...])

    def smem2():
        return pl.BlockSpec(memory_space=pltpu.SMEM)

    def rows():
        return pl.BlockSpec((BLK, D), lambda i: (i, 0))

    def g_rows():
        return pl.BlockSpec((BLK, D), lambda i: (i + g_row_off // BLK, 0))

    def whole(shape):
        return pl.BlockSpec(shape, lambda i: tuple(0 for _ in shape))

    in_specs = [
        smem2(), smem2(), smem2(), smem2(), smem2(),
        rows(), rows(), rows(), rows(), g_rows(), g_rows(),
        whole((D, D)), whole((1, D)), whole((1, D)), whole((1, D)),
        whole((D, D)), whole((1, D)),
        whole((D, D)), whole((1, D)), whole((1, D)), whole((1, D)),
        whole((D, D)), whole((1, D)),
        whole((D, NT)), whole((1, NT)),
    ]
    out_specs = [rows(), rows(), rows(), rows(), rows(),
                 pl.BlockSpec((BLK, NT), lambda i: (i, 0))]
    out_shape = [jax.ShapeDtypeStruct((NU, D), jnp.float32)] * 5 + [
        jax.ShapeDtypeStruct((NU, NT), jnp.float32)]

    return pl.pallas_call(
        body, grid=grid, in_specs=in_specs, out_specs=out_specs,
        out_shape=out_shape,
        compiler_params=pltpu.CompilerParams(
            dimension_semantics=("arbitrary",)),
    )


def _dense_side(u_w, t_w, view_w, lg_w, call_init, call_prop, tag_init,
                tag_prop, g_init, g_prop, g_w,
                mlp_l, mlp_g, pred_w, pred_b, g_row_off):
    fn = _make_dense_side(g_row_off)

    def r1(a):
        return a.reshape(1, -1)

    return fn(u_w, t_w, view_w, lg_w, g_w,
              call_init, call_prop, tag_init, tag_prop, g_init, g_prop,
              mlp_l['w1'], r1(mlp_l['b1']), r1(mlp_l['g']), r1(mlp_l['be']),
              mlp_l['w2'], r1(mlp_l['b2']),
              mlp_g['w1'], r1(mlp_g['b1']), r1(mlp_g['g']), r1(mlp_g['be']),
              mlp_g['w2'], r1(mlp_g['b2']),
              pred_w, r1(pred_b))


def kernel(adj_m_c1_idx, adj_m_c1_val, adj_m_c2_idx, adj_m_c2_val,
           adj_a_c1_idx, adj_a_c1_val, adj_a_c2_idx, adj_a_c2_val,
           adj_m_t1_idx, adj_m_t1_val, adj_m_t2_idx, adj_m_t2_val,
           adj_a_t1_idx, adj_a_t1_val, adj_a_t2_idx, adj_a_t2_val,
           global_1_idx, global_1_val, global_2_idx, global_2_val,
           mashup_call_w, api_call_w, mashup_tag_w, api_tag_w, global_w,
           u_weights, i_weights, m_t_weights, a_t_weights, global_weights,
           mashup_view_weights, api_view_weights, mashup_l_g_weights,
           api_l_g_weights, mlp_mashup_local, mlp_mashup_global,
           mlp_api_local, mlp_api_global, pred_mashup_w, pred_mashup_b,
           pred_api_w, pred_api_b):
    uw = jax.nn.softmax(u_weights)
    iw = jax.nn.softmax(i_weights)
    mtw = jax.nn.softmax(m_t_weights)
    atw = jax.nn.softmax(a_t_weights)
    gw = jax.nn.softmax(global_weights)
    mvw = jax.nn.softmax(mashup_view_weights)
    avw = jax.nn.softmax(api_view_weights)
    mlg = jax.nn.softmax(mashup_l_g_weights)
    alg = jax.nn.softmax(api_l_g_weights)

    xc_mc = _chunked(mashup_call_w)
    xc_ac = _chunked(api_call_w)
    xc_mt = _chunked(mashup_tag_w)
    xc_at = _chunked(api_tag_w)
    xc_g = _chunked(global_w)

    t_mc = _spmm(adj_m_c2_idx, adj_m_c2_val, xc_mc, NI, NU)
    t_ac = _spmm(adj_a_c2_idx, adj_a_c2_val, xc_ac, NU, NI)
    t_mt = _spmm(adj_m_t2_idx, adj_m_t2_val, xc_mt, NT, NU)
    t_at = _spmm(adj_a_t2_idx, adj_a_t2_val, xc_at, NT, NI)
    t_g = _spmm(global_1_idx, global_1_val, xc_g, NG, NG)

    def _unchunk(t, n):
        return jnp.transpose(t, (1, 0, 2)).reshape(n, D)

    p_mc = _unchunk(_spmm(adj_m_c1_idx, adj_m_c1_val, t_mc, NU, NI), NU)
    p_ac = _unchunk(_spmm(adj_a_c1_idx, adj_a_c1_val, t_ac, NI, NU), NI)
    p_mt = _unchunk(_spmm(adj_m_t1_idx, adj_m_t1_val, t_mt, NU, NT), NU)
    p_at = _unchunk(_spmm(adj_a_t1_idx, adj_a_t1_val, t_at, NI, NT), NI)
    p_g = _unchunk(_spmm(global_2_idx, global_2_val, t_g, NG, NG), NG)

    m_final, m_cp, m_tp, m_lp, m_gp, m_logit = _dense_side(
        uw, mtw, mvw, mlg, mashup_call_w, p_mc, mashup_tag_w, p_mt,
        global_w, p_g, gw,
        mlp_mashup_local, mlp_mashup_global, pred_mashup_w, pred_mashup_b, 0)
    a_final, a_cp, a_tp, a_lp, a_gp, a_logit = _dense_side(
        iw, atw, avw, alg, api_call_w, p_ac, api_tag_w, p_at,
        global_w, p_g, gw,
        mlp_api_local, mlp_api_global, pred_api_w, pred_api_b, NU)

    return (m_final, a_final, m_cp, m_tp, a_cp, a_tp, m_lp, a_lp,
            m_gp, a_gp, m_logit, a_logit)


# speedup vs baseline: 1.8104x; 1.8104x over previous
import functools

import jax
import jax.numpy as jnp
from jax import lax
from jax.experimental import pallas as pl
from jax.experimental.pallas import tpu as pltpu
from jax.experimental.pallas import tpu_sc as plsc

NU = 10000
NI = 10000
NT = 2000
NG = NU + NI + NT
D = 768

NC = 2
NS = 16
EB = 128
_GDN = lax.GatherDimensionNumbers(offset_dims=(), collapsed_slice_dims=(0,),
                                  start_index_map=(0,))
ZROWS = 125


def _chunked(x, cw):
    n = x.shape[0]
    return jnp.transpose(x.reshape(n, D // cw, cw), (1, 0, 2))


def _pad_edges(idx, val, n_rows, n_cols):
    nnz = val.shape[0]
    e_pad = -(-nnz // (NS * EB * 2)) * (NS * EB * 2)
    pad = e_pad - nnz
    ar = jnp.arange(pad, dtype=jnp.int32)
    rows = jnp.concatenate([idx[0], ar % n_rows])
    cols = jnp.concatenate([idx[1], ar % n_cols])
    vals = jnp.concatenate([val, jnp.zeros((pad,), jnp.float32)])
    return rows, cols, vals


@functools.lru_cache(maxsize=None)
def _make_spmm(n_rows, n_cols, e_pad, cw):
    per_tile = e_pad // NS
    n_batches = per_tile // EB
    nchunk = D // cw
    chunks_per_sc = nchunk // NC
    stripe = -(-n_rows // (8 * NS)) * 8
    stripe_last = n_rows - (NS - 1) * stripe
    assert stripe_last > 0 and stripe_last % 8 == 0
    mesh = plsc.VectorSubcoreMesh(core_axis_name="c", subcore_axis_name="s",
                                  num_cores=NC, num_subcores=NS)

    @functools.partial(
        pl.kernel,
        out_type=jax.ShapeDtypeStruct((nchunk, n_rows, cw), jnp.float32),
        mesh=mesh,
        scratch_types=[
            pltpu.VMEM_SHARED((n_rows, cw), jnp.float32),
            pltpu.VMEM((n_batches, EB), jnp.int32),
            pltpu.VMEM((n_batches, EB), jnp.int32),
            pltpu.VMEM((per_tile + 16,), jnp.float32),
            pltpu.VMEM((EB, cw), jnp.float32),
            pltpu.VMEM((EB, cw), jnp.float32),
            pltpu.SemaphoreType.DMA,
            pltpu.SemaphoreType.DMA,
            pltpu.SemaphoreType.DMA,
            pltpu.SemaphoreType.DMA,
        ],
        compiler_params=pltpu.CompilerParams(use_tc_tiling_on_sc=False),
    )
    def spmm(x_hbm, rows_hbm, cols_hbm, vals_hbm, zeros_hbm, out_hbm,
             acc, colsl, rowsl, valsl, gbuf0, gbuf1,
             semg0, semg1, sems0, sems1):
        cid = lax.axis_index("c")
        sid = lax.axis_index("s")
        rbase = sid * stripe

        tb0 = sid * n_batches
        pltpu.sync_copy(cols_hbm.at[pl.ds(tb0, n_batches), :], colsl)
        pltpu.sync_copy(rows_hbm.at[pl.ds(tb0, n_batches), :], rowsl)
        pltpu.sync_copy(vals_hbm.at[pl.ds(sid * per_tile, per_tile)],
                        valsl.at[pl.ds(0, per_tile)])

        gbuf = [gbuf0, gbuf1]
        semg = [semg0, semg1]
        sems = [sems0, sems1]

        @pl.loop(0, chunks_per_sc)
        def _(ci):
            c = cid * chunks_per_sc + ci

            @pl.when(sid < NS - 1)
            def _():
                pltpu.sync_copy(zeros_hbm.at[pl.ds(0, stripe), :],
                                acc.at[pl.ds(rbase, stripe), :])

            @pl.when(sid == NS - 1)
            def _():
                pltpu.sync_copy(zeros_hbm.at[pl.ds(0, stripe_last), :],
                                acc.at[pl.ds(rbase, stripe_last), :])

            plsc.subcore_barrier()

            gd = [None, None]
            sd = [None, None]
            gd[0] = pltpu.async_copy(x_hbm.at[c].at[colsl.at[0]], gbuf[0],
                                     semg[0])
            for b in range(n_batches):
                k = b % 2
                nk = 1 - k
                if b + 1 < n_batches:
                    if b >= 1:
                        sd[nk].wait()
                    gd[nk] = pltpu.async_copy(
                        x_hbm.at[c].at[colsl.at[b + 1]], gbuf[nk], semg[nk])
                gd[k].wait()

                @plsc.parallel_loop(0, EB, unroll=4)
                def _(e):
                    v = valsl[pl.ds(b * EB + e, 16)][0]
                    for j in range(cw // 16):
                        gbuf[k][e, pl.ds(j * 16, 16)] = (
                            gbuf[k][e, pl.ds(j * 16, 16)] * v)

                sd[k] = pltpu.async_copy(gbuf[k], acc.at[rowsl.at[b]],
                                         sems[k], add=True)
            sd[(n_batches - 1) % 2].wait()
            if n_batches >= 2:
                sd[n_batches % 2].wait()

            plsc.subcore_barrier()

            @pl.when(sid < NS - 1)
            def _():
                pltpu.sync_copy(acc.at[pl.ds(rbase, stripe), :],
                                out_hbm.at[c, pl.ds(rbase, stripe), :])

            @pl.when(sid == NS - 1)
            def _():
                pltpu.sync_copy(acc.at[pl.ds(rbase, stripe_last), :],
                                out_hbm.at[c, pl.ds(rbase, stripe_last), :])

            plsc.subcore_barrier()

    return spmm


def _spmm(idx, val, x_chunks, n_rows, n_cols, cw):
    rows, cols, vals = _pad_edges(idx, val, n_rows, n_cols)
    e_pad = int(vals.shape[0])
    stripe = -(-n_rows // (8 * NS)) * 8
    zeros = jnp.zeros((stripe, cw), jnp.float32)
    fn = _make_spmm(n_rows, n_cols, e_pad, cw)
    return fn(x_chunks, rows.reshape(e_pad // EB, EB),
              cols.reshape(e_pad // EB, EB), vals, zeros)


def _gelu_exact(x):
    return 0.5 * x * (1.0 + lax.erf(x * (2.0 ** -0.5)))


def _mlp_block(x, w1, b1, g, be, w2, b2):
    h = jnp.dot(x, w1, preferred_element_type=jnp.float32) + b1
    m = jnp.mean(h, axis=1, keepdims=True)
    v = jnp.mean((h - m) * (h - m), axis=1, keepdims=True)
    h = (h - m) * lax.rsqrt(v + 1e-5) * g + be
    h = _gelu_exact(h)
    return jnp.dot(h, w2, preferred_element_type=jnp.float32) + b2


BLK = 200
NBLK = NU // BLK


@functools.lru_cache(maxsize=None)
def _make_dense_side(g_row_off):
    grid = (NBLK,)

    def body(uw_s, tw_s, vw_s, lgw_s, gw_s,
             ci_ref, cp_ref, ti_ref, tp_ref, gi_ref, gp_ref,
             w1l, b1l, gl, bel, w2l, b2l,
             w1g, b1g, gg, beg, w2g, b2g,
             pw, pb,
             out_final, out_cp, out_tp, out_lp, out_gp, out_logit):
        call_emb = uw_s[0] * ci_ref[...] + uw_s[1] * cp_ref[...]
        tag_emb = tw_s[0] * ti_ref[...] + tw_s[1] * tp_ref[...]
        g_emb = gw_s[0] * gi_ref[...] + gw_s[1] * gp_ref[...]
        mixed = vw_s[0] * call_emb + vw_s[1] * tag_emb
        final = lgw_s[0] * mixed + lgw_s[1] * g_emb
        out_final[...] = final
        xs = jnp.concatenate([call_emb, tag_emb, mixed], axis=0)
        ys = _mlp_block(xs, w1l[...], b1l[...], gl[...], bel[...],
                        w2l[...], b2l[...])
        out_cp[...] = ys[:BLK]
        out_tp[...] = ys[BLK:2 * BLK]
        out_lp[...] = ys[2 * BLK:]
        out_gp[...] = _mlp_block(g_emb, w1g[...], b1g[...], gg[...], beg[...],
                                 w2g[...], b2g[...])
        out_logit[...] = (jnp.dot(final, pw[...], preferred_element_type=jnp.float32)
                          + pb[...])

    def smem2():
        return pl.BlockSpec(memory_space=pltpu.SMEM)

    def rows():
        return pl.BlockSpec((BLK, D), lambda i: (i, 0))

    def g_rows():
        return pl.BlockSpec((BLK, D), lambda i: (i + g_row_off // BLK, 0))

    def whole(shape):
        return pl.BlockSpec(shape, lambda i: tuple(0 for _ in shape))

    in_specs = [
        smem2(), smem2(), smem2(), smem2(), smem2(),
        rows(), rows(), rows(), rows(), g_rows(), g_rows(),
        whole((D, D)), whole((1, D)), whole((1, D)), whole((1, D)),
        whole((D, D)), whole((1, D)),
        whole((D, D)), whole((1, D)), whole((1, D)), whole((1, D)),
        whole((D, D)), whole((1, D)),
        whole((D, NT)), whole((1, NT)),
    ]
    out_specs = [rows(), rows(), rows(), rows(), rows(),
                 pl.BlockSpec((BLK, NT), lambda i: (i, 0))]
    out_shape = [jax.ShapeDtypeStruct((NU, D), jnp.float32)] * 5 + [
        jax.ShapeDtypeStruct((NU, NT), jnp.float32)]

    return pl.pallas_call(
        body, grid=grid, in_specs=in_specs, out_specs=out_specs,
        out_shape=out_shape,
        compiler_params=pltpu.CompilerParams(
            dimension_semantics=("arbitrary",)),
    )


def _dense_side(u_w, t_w, view_w, lg_w, call_init, call_prop, tag_init,
                tag_prop, g_init, g_prop, g_w,
                mlp_l, mlp_g, pred_w, pred_b, g_row_off):
    fn = _make_dense_side(g_row_off)

    def r1(a):
        return a.reshape(1, -1)

    return fn(u_w, t_w, view_w, lg_w, g_w,
              call_init, call_prop, tag_init, tag_prop, g_init, g_prop,
              mlp_l['w1'], r1(mlp_l['b1']), r1(mlp_l['g']), r1(mlp_l['be']),
              mlp_l['w2'], r1(mlp_l['b2']),
              mlp_g['w1'], r1(mlp_g['b1']), r1(mlp_g['g']), r1(mlp_g['be']),
              mlp_g['w2'], r1(mlp_g['b2']),
              pred_w, r1(pred_b))


def kernel(adj_m_c1_idx, adj_m_c1_val, adj_m_c2_idx, adj_m_c2_val,
           adj_a_c1_idx, adj_a_c1_val, adj_a_c2_idx, adj_a_c2_val,
           adj_m_t1_idx, adj_m_t1_val, adj_m_t2_idx, adj_m_t2_val,
           adj_a_t1_idx, adj_a_t1_val, adj_a_t2_idx, adj_a_t2_val,
           global_1_idx, global_1_val, global_2_idx, global_2_val,
           mashup_call_w, api_call_w, mashup_tag_w, api_tag_w, global_w,
           u_weights, i_weights, m_t_weights, a_t_weights, global_weights,
           mashup_view_weights, api_view_weights, mashup_l_g_weights,
           api_l_g_weights, mlp_mashup_local, mlp_mashup_global,
           mlp_api_local, mlp_api_global, pred_mashup_w, pred_mashup_b,
           pred_api_w, pred_api_b):
    uw = jax.nn.softmax(u_weights)
    iw = jax.nn.softmax(i_weights)
    mtw = jax.nn.softmax(m_t_weights)
    atw = jax.nn.softmax(a_t_weights)
    gw = jax.nn.softmax(global_weights)
    mvw = jax.nn.softmax(mashup_view_weights)
    avw = jax.nn.softmax(api_view_weights)
    mlg = jax.nn.softmax(mashup_l_g_weights)
    alg = jax.nn.softmax(api_l_g_weights)

    xc_mc = _chunked(mashup_call_w, 96)
    xc_ac = _chunked(api_call_w, 96)
    xc_mt = _chunked(mashup_tag_w, 96)
    xc_at = _chunked(api_tag_w, 96)
    xc_g = _chunked(global_w, 48)

    t_mc = _spmm(adj_m_c2_idx, adj_m_c2_val, xc_mc, NI, NU, 96)
    t_ac = _spmm(adj_a_c2_idx, adj_a_c2_val, xc_ac, NU, NI, 96)
    t_mt = _spmm(adj_m_t2_idx, adj_m_t2_val, xc_mt, NT, NU, 96)
    t_at = _spmm(adj_a_t2_idx, adj_a_t2_val, xc_at, NT, NI, 96)
    t_g = _spmm(global_1_idx, global_1_val, xc_g, NG, NG, 48)

    def _unchunk(t, n):
        return jnp.transpose(t, (1, 0, 2)).reshape(n, D)

    p_mc = _unchunk(_spmm(adj_m_c1_idx, adj_m_c1_val, t_mc, NU, NI, 96), NU)
    p_ac = _unchunk(_spmm(adj_a_c1_idx, adj_a_c1_val, t_ac, NI, NU, 96), NI)
    p_mt = _unchunk(_spmm(adj_m_t1_idx, adj_m_t1_val, t_mt, NU, NT, 96), NU)
    p_at = _unchunk(_spmm(adj_a_t1_idx, adj_a_t1_val, t_at, NI, NT, 96), NI)
    p_g = _unchunk(_spmm(global_2_idx, global_2_val, t_g, NG, NG, 48), NG)

    m_final, m_cp, m_tp, m_lp, m_gp, m_logit = _dense_side(
        uw, mtw, mvw, mlg, mashup_call_w, p_mc, mashup_tag_w, p_mt,
        global_w, p_g, gw,
        mlp_mashup_local, mlp_mashup_global, pred_mashup_w, pred_mashup_b, 0)
    a_final, a_cp, a_tp, a_lp, a_gp, a_logit = _dense_side(
        iw, atw, avw, alg, api_call_w, p_ac, api_tag_w, p_at,
        global_w, p_g, gw,
        mlp_api_local, mlp_api_global, pred_api_w, pred_api_b, NU)

    return (m_final, a_final, m_cp, m_tp, a_cp, a_tp, m_lp, a_lp,
            m_gp, a_gp, m_logit, a_logit)
